# baseline (speedup 1.0000x reference)
import functools

import jax
import jax.numpy as jnp
from jax import lax
from jax.experimental import pallas as pl
from jax.experimental.pallas import tpu as pltpu
from jax.experimental.pallas import tpu_sc as plsc


def _sc_gather(gammas, ts):
    mesh = plsc.VectorSubcoreMesh(core_axis_name="c", subcore_axis_name="s")

    @functools.partial(
        pl.kernel,
        mesh=mesh,
        out_type=jax.ShapeDtypeStruct((32,), jnp.float32),
        scratch_types=[
            pltpu.VMEM((16,), jnp.int32),
            pltpu.VMEM((16,), jnp.float32),
            pltpu.SemaphoreType.DMA,
        ],
    )
    def k(g_hbm, t_hbm, out_hbm, idx_v, rows_v, sem):
        wid = lax.axis_index("s") * 2 + lax.axis_index("c")

        @pl.when(wid < 2)
        def _():
            base = wid * 16
            pltpu.sync_copy(t_hbm.at[pl.ds(base, 16)], idx_v)
            pltpu.async_copy(g_hbm.at[idx_v], rows_v, sem).wait()
            pltpu.sync_copy(rows_v, out_hbm.at[pl.ds(base, 16)])

    return k(gammas, ts)


def _tc_body(ts_ref, gam_ref, x_ref, n_ref, o_ref):
    b = pl.program_id(0)
    g = gam_ref[ts_ref[b]]
    o_ref[...] = jnp.sqrt(g) * x_ref[...] + jnp.sqrt(1.0 - g) * n_ref[...]


_ROWS_PER_BLOCK = 128


def kernel(x_start, timesteps, noise, gammas):
    B, C, H, W = x_start.shape
    ts = timesteps.reshape(B).astype(jnp.int32)

    lanes = 1024
    rows = (C * H * W) // lanes
    x3 = x_start.reshape(B, rows, lanes)
    n3 = noise.reshape(B, rows, lanes)

    rb = _ROWS_PER_BLOCK
    grid_spec = pltpu.PrefetchScalarGridSpec(
        num_scalar_prefetch=2,
        grid=(B, rows // rb),
        in_specs=[
            pl.BlockSpec((1, rb, lanes), lambda b, r, t, g: (b, r, 0)),
            pl.BlockSpec((1, rb, lanes), lambda b, r, t, g: (b, r, 0)),
        ],
        out_specs=pl.BlockSpec((1, rb, lanes), lambda b, r, t, g: (b, r, 0)),
    )
    out3 = pl.pallas_call(
        _tc_body,
        grid_spec=grid_spec,
        out_shape=jax.ShapeDtypeStruct((B, rows, lanes), jnp.float32),
    )(ts, gammas.astype(jnp.float32), x3, n3)
    return out3.reshape(B, C, H, W)


# speedup vs baseline: 1.1762x; 1.1762x over previous
import jax
import jax.numpy as jnp
from jax import lax
from jax.experimental import pallas as pl
from jax.experimental.pallas import tpu as pltpu

_NBUF = 4


def _body(ts_ref, gam_ref, x_hbm, n_hbm, o_hbm, xb, nb, ob, xsem, nsem, osem):
    nchunks = x_hbm.shape[0]

    def start_in(i, slot):
        pltpu.make_async_copy(x_hbm.at[i], xb.at[slot], xsem.at[slot]).start()
        pltpu.make_async_copy(n_hbm.at[i], nb.at[slot], nsem.at[slot]).start()

    for i in range(_NBUF):
        start_in(i, i)

    def step(i, _):
        slot = lax.rem(i, _NBUF)
        pltpu.make_async_copy(x_hbm.at[i], xb.at[slot], xsem.at[slot]).wait()
        pltpu.make_async_copy(n_hbm.at[i], nb.at[slot], nsem.at[slot]).wait()

        @pl.when(i >= _NBUF)
        def _():
            pltpu.make_async_copy(
                ob.at[slot], o_hbm.at[i - _NBUF], osem.at[slot]
            ).wait()

        g = gam_ref[ts_ref[i]]
        sa = jnp.sqrt(g)
        sb = jnp.sqrt(1.0 - g)
        ob[slot] = sa * xb[slot] + sb * nb[slot]
        pltpu.make_async_copy(ob.at[slot], o_hbm.at[i], osem.at[slot]).start()

        @pl.when(i + _NBUF < nchunks)
        def _():
            start_in(i + _NBUF, slot)

        return 0

    lax.fori_loop(0, nchunks, step, 0)

    def drain(i, _):
        slot = lax.rem(i, _NBUF)
        pltpu.make_async_copy(ob.at[slot], o_hbm.at[i], osem.at[slot]).wait()
        return 0

    lax.fori_loop(nchunks - _NBUF, nchunks, drain, 0)


def kernel(x_start, timesteps, noise, gammas):
    B, C, H, W = x_start.shape
    ts = timesteps.reshape(B).astype(jnp.int32)

    lanes = 1024
    rows = (C * H * W) // lanes
    x3 = x_start.reshape(B, rows, lanes)
    n3 = noise.reshape(B, rows, lanes)

    out3 = pl.pallas_call(
        _body,
        grid=(),
        in_specs=[
            pl.BlockSpec(memory_space=pltpu.SMEM),
            pl.BlockSpec(memory_space=pltpu.SMEM),
            pl.BlockSpec(memory_space=pltpu.HBM),
            pl.BlockSpec(memory_space=pltpu.HBM),
        ],
        out_specs=pl.BlockSpec(memory_space=pltpu.HBM),
        scratch_shapes=[
            pltpu.VMEM((_NBUF, rows, lanes), jnp.float32),
            pltpu.VMEM((_NBUF, rows, lanes), jnp.float32),
            pltpu.VMEM((_NBUF, rows, lanes), jnp.float32),
            pltpu.SemaphoreType.DMA((_NBUF,)),
            pltpu.SemaphoreType.DMA((_NBUF,)),
            pltpu.SemaphoreType.DMA((_NBUF,)),
        ],
        out_shape=jax.ShapeDtypeStruct((B, rows, lanes), jnp.float32),
    )(ts, gammas.astype(jnp.float32), x3, n3)
    return out3.reshape(B, C, H, W)


# speedup vs baseline: 5.0788x; 4.3180x over previous
import jax
import jax.numpy as jnp
from jax.experimental import pallas as pl
from jax.experimental.pallas import tpu as pltpu


def _tc_body(ts_ref, gam_ref, x_ref, n_ref, o_ref):
    b = pl.program_id(0)
    g = gam_ref[ts_ref[b]]
    o_ref[...] = jnp.sqrt(g) * x_ref[...] + jnp.sqrt(1.0 - g) * n_ref[...]


def kernel(x_start, timesteps, noise, gammas):
    B, C, H, W = x_start.shape
    ts = timesteps.reshape(B).astype(jnp.int32)

    grid_spec = pltpu.PrefetchScalarGridSpec(
        num_scalar_prefetch=2,
        grid=(B,),
        in_specs=[
            pl.BlockSpec((1, C, H, W), lambda b, t, g: (b, 0, 0, 0)),
            pl.BlockSpec((1, C, H, W), lambda b, t, g: (b, 0, 0, 0)),
        ],
        out_specs=pl.BlockSpec((1, C, H, W), lambda b, t, g: (b, 0, 0, 0)),
    )
    return pl.pallas_call(
        _tc_body,
        grid_spec=grid_spec,
        out_shape=jax.ShapeDtypeStruct((B, C, H, W), jnp.float32),
    )(ts, gammas.astype(jnp.float32), x_start, noise)
